# baseline (speedup 1.0000x reference)
import jax
import jax.numpy as jnp
import numpy as np
from jax.experimental import pallas as pl

_D = 128
_H = 8
_SBLK = 2048
_LEVELS = 3.0
_RCP = float(np.float32(1.0) / np.float32(_LEVELS))


def _body(kv_ref, r_ref, cb_ref, out_ref):
    x = kv_ref[0, 0]
    s = jnp.max(jnp.abs(x), axis=-1, keepdims=True) + 1e-8
    m = jnp.round(jnp.clip(x / s, -1.0, 1.0) * _LEVELS)
    step = jnp.float32(_RCP) * s
    q = m * step
    resid = x - q
    alpha = jnp.mean(jnp.abs(resid), axis=-1, keepdims=True)
    a_recon = q + alpha * jnp.sign(resid)
    out_ref[0, 0] = a_recon

    others = kv_ref[0, 1:]
    delta = others - a_recon[None]
    dn = jnp.sqrt(jnp.sum(delta * delta, axis=-1, keepdims=True))
    inv = 1.0 / (dn + 1e-8)
    dnorm = (delta * inv).reshape((_H - 1) * _SBLK, _D)

    rb = r_ref[...].astype(jnp.bfloat16)
    y = jax.lax.dot_general(dnorm.astype(jnp.bfloat16), rb,
                            (((1,), (1,)), ((), ())),
                            preferred_element_type=jnp.float32)
    cb0 = cb_ref[0, 0]
    cb1 = cb_ref[0, 1]
    yq = jnp.where(y <= 0.5 * (cb0 + cb1), cb0, cb1)
    dr = jax.lax.dot_general(yq.astype(jnp.bfloat16), rb,
                             (((1,), (0,)), ((), ())),
                             preferred_element_type=jnp.float32)
    out_ref[0, 1:] = a_recon[None] + dr.reshape(_H - 1, _SBLK, _D) * dn


def kernel(kv_states, R, codebook):
    b, h, seq, d = kv_states.shape
    cb = codebook.reshape(1, 2)
    grid = (b, seq // _SBLK)
    return pl.pallas_call(
        _body,
        grid=grid,
        in_specs=[
            pl.BlockSpec((1, h, _SBLK, d), lambda i, j: (i, 0, j, 0)),
            pl.BlockSpec((d, d), lambda i, j: (0, 0)),
            pl.BlockSpec((1, 2), lambda i, j: (0, 0)),
        ],
        out_specs=pl.BlockSpec((1, h, _SBLK, d), lambda i, j: (i, 0, j, 0)),
        out_shape=jax.ShapeDtypeStruct((b, h, seq, d), kv_states.dtype),
    )(kv_states, R, cb)


# speedup vs baseline: 1.0003x; 1.0003x over previous
import jax
import jax.numpy as jnp
import numpy as np
from jax.experimental import pallas as pl

_D = 128
_H = 8
_SBLK = 2048
_LEVELS = 3.0
_RCP = float(np.float32(1.0) / np.float32(_LEVELS))


def _body(kv_ref, r_ref, cb_ref, out_ref):
    x = kv_ref[0, 0]
    s = jnp.max(jnp.abs(x), axis=-1, keepdims=True) + 1e-8
    m = jnp.round(jnp.clip(x / s, -1.0, 1.0) * _LEVELS)
    step = jnp.float32(_RCP) * s
    q = m * step
    resid = x - q
    alpha = jnp.mean(jnp.abs(resid), axis=-1, keepdims=True)
    a_recon = q + alpha * jnp.sign(resid)
    out_ref[0, 0] = a_recon

    others = kv_ref[0, 1:]
    delta = others - a_recon[None]
    dn = jnp.sqrt(jnp.sum(delta * delta, axis=-1, keepdims=True))
    dnorm = (delta / (dn + 1e-8)).reshape((_H - 1) * _SBLK, _D)

    rb = r_ref[...].astype(jnp.bfloat16)
    y = jax.lax.dot_general(dnorm.astype(jnp.bfloat16), rb,
                            (((1,), (1,)), ((), ())),
                            preferred_element_type=jnp.float32)
    cb0 = cb_ref[0, 0]
    cb1 = cb_ref[0, 1]
    yq = jnp.where(y <= 0.5 * (cb0 + cb1), cb0, cb1)
    dr = jax.lax.dot_general(yq.astype(jnp.bfloat16), rb,
                             (((1,), (0,)), ((), ())),
                             preferred_element_type=jnp.float32)
    out_ref[0, 1:] = a_recon[None] + dr.reshape(_H - 1, _SBLK, _D) * dn


def kernel(kv_states, R, codebook):
    b, h, seq, d = kv_states.shape
    cb = codebook.reshape(1, 2)
    grid = (b, seq // _SBLK)
    return pl.pallas_call(
        _body,
        grid=grid,
        in_specs=[
            pl.BlockSpec((1, h, _SBLK, d), lambda i, j: (i, 0, j, 0)),
            pl.BlockSpec((d, d), lambda i, j: (0, 0)),
            pl.BlockSpec((1, 2), lambda i, j: (0, 0)),
        ],
        out_specs=pl.BlockSpec((1, h, _SBLK, d), lambda i, j: (i, 0, j, 0)),
        out_shape=jax.ShapeDtypeStruct((b, h, seq, d), kv_states.dtype),
    )(kv_states, R, cb)
